# baseline (speedup 1.0000x reference)
import functools

import jax
import jax.numpy as jnp
from jax import lax
from jax.experimental import pallas as pl
from jax.experimental.pallas import tpu as pltpu
from jax.experimental.pallas import tpu_sc as plsc

_LINKAGE = [(15, 13), (13, 11), (16, 14), (14, 12), (11, 12), (5, 11),
            (6, 12), (5, 6), (5, 7), (6, 8), (7, 9), (8, 10), (1, 2),
            (0, 1), (0, 2), (1, 3), (2, 4), (3, 5), (4, 6)]
_PDT = 1.0
_EXPANSION = 0.3
_HW_RATIO = 2.0
_BS, _P, _J, _H, _W = 8, 10, 17, 64, 64
_L = len(_LINKAGE)
_C = 2 * _L
_TS = (_H * _W) // 128
_NL = 16
_TR = _L * 8 + 4
_NP = 10
_BIG = 3.0e38
_MAGIC = 12582912.0


def _round_ne(x):
    return (x + _MAGIC) - _MAGIC


def _rsqrt_newton(n2):
    i = lax.bitcast_convert_type(n2, jnp.int32)
    i = jnp.int32(0x5F3759DF) - lax.shift_right_arithmetic(i, 1)
    y = lax.bitcast_convert_type(i, jnp.float32)
    for _ in range(3):
        y = y * (1.5 - 0.5 * n2 * y * y)
    return y


def _sc_body(jt_hbm, t_hbm, jv, t_v):
    w = lax.axis_index("c") * 16 + lax.axis_index("s")

    @pl.when(w < _BS)
    def _():
        b = w
        pltpu.sync_copy(jt_hbm.at[b], jv)

        lane = lax.broadcasted_iota(jnp.int32, (_NL,), 0)
        one = jnp.ones((_NL,), jnp.float32)

        tlx = one * _BIG
        tly = one * _BIG
        brx = -one * _BIG
        bry = -one * _BIG
        visp = lane < 0
        for j in range(_J):
            xj = jv[j, 0]
            yj = jv[j, 1]
            vj = jv[j, 2]
            inv = vj <= 0.0
            tlx = jnp.minimum(tlx, jnp.where(inv, _BIG, xj))
            tly = jnp.minimum(tly, jnp.where(inv, _BIG, yj))
            brx = jnp.maximum(brx, jnp.where(inv, -_BIG, xj))
            bry = jnp.maximum(bry, jnp.where(inv, -_BIG, yj))
            visp = visp | (vj > 0.0)
        whx = brx - tlx
        why = bry - tly
        whx = jnp.where(whx < 1.0, 1.0, whx)
        why = jnp.where(why < 1.0, 1.0, why)
        ctx = 0.5 * (brx + tlx)
        cty = 0.5 * (bry + tly)
        whx2 = jnp.maximum(whx, why / _HW_RATIO)
        why2 = jnp.maximum(why, whx / _HW_RATIO)
        exp = jnp.float32(0.5 + _EXPANSION)
        t_v[_L * 8 + 0] = jnp.where(visp, _round_ne(ctx - exp * whx2), _BIG)
        t_v[_L * 8 + 1] = jnp.where(visp, _round_ne(ctx + exp * whx2), -_BIG)
        t_v[_L * 8 + 2] = jnp.where(visp, _round_ne(cty - exp * why2), _BIG)
        t_v[_L * 8 + 3] = jnp.where(visp, _round_ne(cty + exp * why2), -_BIG)

        for l, (a, bb) in enumerate(_LINKAGE):
            sx = jv[a, 0]
            sy = jv[a, 1]
            sv = jv[a, 2]
            ex = jv[bb, 0]
            ey = jv[bb, 1]
            ev = jv[bb, 2]
            valid = ((sv > 0.0) & (ev > 0.0) &
                     ((sx != ex) | (sy != ey)))
            vecx = jnp.where(valid, ex - sx, 1.0)
            vecy = jnp.where(valid, ey - sy, 1.0)
            n2 = vecx * vecx + vecy * vecy
            rs = _rsqrt_newton(n2)
            ux = vecx * rs
            uy = vecy * rs
            norm = n2 * rs
            sdot = sx * ux + sy * uy
            scross = sx * uy - sy * ux
            r = l * 8
            t_v[r + 0] = ux
            t_v[r + 1] = uy
            t_v[r + 2] = sdot - _PDT
            t_v[r + 3] = jnp.where(valid, sdot + norm + _PDT, -_BIG)
            t_v[r + 4] = scross - _PDT
            t_v[r + 5] = scross + _PDT
            invis_l = (sv <= 0.0) | (ev <= 0.0)
            t_v[r + 6] = jnp.where(visp & invis_l, 1.0, 0.0)
            t_v[r + 7] = jnp.where(valid, 1.0, 0.0)

        pltpu.sync_copy(t_v, t_hbm.at[b])


@functools.lru_cache(maxsize=1)
def _sc_prep():
    return pl.kernel(
        _sc_body,
        out_type=jax.ShapeDtypeStruct((_BS, _TR, _NL), jnp.float32),
        mesh=plsc.VectorSubcoreMesh(core_axis_name="c",
                                    subcore_axis_name="s",
                                    num_cores=1),
        scratch_types=[
            pltpu.VMEM((_J, 3, _NL), jnp.float32),
            pltpu.VMEM((_TR, _NL), jnp.float32),
        ],
    )


def _tc_body(paf_ref, m_ref, t_ref, out_ref, idx_ref):
    f32 = jnp.float32
    pix = lax.broadcasted_iota(jnp.int32, (_TS, 128), 0) * 128 + \
        lax.broadcasted_iota(jnp.int32, (_TS, 128), 1)
    yf = (pix // _W).astype(f32)
    xf = (pix % _W).astype(f32)

    bits = jnp.zeros((_TS, 128), jnp.int32)
    for p in range(_P):
        inb = ((xf >= t_ref[0, _L * 8 + 0, p]) &
               (xf <= t_ref[0, _L * 8 + 1, p]) &
               (yf >= t_ref[0, _L * 8 + 2, p]) &
               (yf <= t_ref[0, _L * 8 + 3, p]))
        bits = bits + jnp.where(inb, jnp.int32(1 << p), jnp.int32(0))

    mask_t = m_ref[0]
    lacc = jnp.zeros((_TS, 128), f32)
    for l in range(_L):
        r = l * 8
        n = jnp.int32(0)
        for p in range(_P):
            idx_ref[n] = jnp.int32(p)
            n = n + jnp.where(t_ref[0, r + 7, p] > 0.0,
                              jnp.int32(1), jnp.int32(0))

        def _pbody(k, carry, r=r):
            count, txn, tyn = carry
            p = idx_ref[k]
            ux_s = t_ref[0, r + 0, p]
            uy_s = t_ref[0, r + 1, p]
            tt = xf * ux_s + yf * uy_s
            cc = xf * uy_s - yf * ux_s
            m = ((tt >= t_ref[0, r + 2, p]) & (tt <= t_ref[0, r + 3, p]) &
                 (cc >= t_ref[0, r + 4, p]) & (cc <= t_ref[0, r + 5, p]))
            return (count + jnp.where(m, 1.0, 0.0),
                    txn + jnp.where(m, ux_s, 0.0),
                    tyn + jnp.where(m, uy_s, 0.0))

        zero = jnp.zeros((_TS, 128), f32)
        count, txn, tyn = lax.fori_loop(0, n, _pbody, (zero, zero, zero))

        act = jnp.int32(0)
        for p in range(_P):
            act = act + jnp.where(t_ref[0, r + 6, p] > 0.0,
                                  jnp.int32(1 << p), jnp.int32(0))
        bad = (bits & act) != 0
        paf_lw = jnp.minimum(mask_t, jnp.where(bad, 0.0, 1.0))
        div = jnp.maximum(count, 1.0)
        tx = txn / div
        ty = tyn / div
        lw = jnp.where(count > 0.0, 1.0, paf_lw)
        dx = paf_ref[0, 2 * l] - tx
        dy = paf_ref[0, 2 * l + 1] - ty
        lacc = lacc + (dx * dx + dy * dy) * lw

    out_ref[0, 0] = jnp.full((128,), jnp.sum(lacc) / (_C * _H * _W),
                             jnp.float32)


def kernel(paf_pred, jointsXYV, mask):
    jt = jnp.transpose(jointsXYV, (0, 2, 3, 1))
    jt = jnp.pad(jt, ((0, 0), (0, 0), (0, 0), (0, _NL - _P)),
                 constant_values=-1.0)
    t = _sc_prep()(jt)

    paf2 = paf_pred.reshape(_BS, _C, _TS, 128)
    mask2 = mask.reshape(_BS, _TS, 128)
    out = pl.pallas_call(
        _tc_body,
        grid=(_BS,),
        in_specs=[
            pl.BlockSpec((1, _C, _TS, 128), lambda b: (b, 0, 0, 0)),
            pl.BlockSpec((1, _TS, 128), lambda b: (b, 0, 0)),
            pl.BlockSpec((1, _TR, _NL), lambda b: (b, 0, 0),
                         memory_space=pltpu.SMEM),
        ],
        out_specs=pl.BlockSpec((1, 1, 128), lambda b: (b, 0, 0)),
        out_shape=jax.ShapeDtypeStruct((_BS, 1, 128), jnp.float32),
        scratch_shapes=[pltpu.SMEM((16,), jnp.int32)],
    )(paf2, mask2, t)
    return out[:, 0, 0]


# speedup vs baseline: 1.1977x; 1.1977x over previous
import functools

import jax
import jax.numpy as jnp
from jax import lax
from jax.experimental import pallas as pl
from jax.experimental.pallas import tpu as pltpu
from jax.experimental.pallas import tpu_sc as plsc

_LINKAGE = [(15, 13), (13, 11), (16, 14), (14, 12), (11, 12), (5, 11),
            (6, 12), (5, 6), (5, 7), (6, 8), (7, 9), (8, 10), (1, 2),
            (0, 1), (0, 2), (1, 3), (2, 4), (3, 5), (4, 6)]
_PDT = 1.0
_EXPANSION = 0.3
_HW_RATIO = 2.0
_BS, _P, _J, _H, _W = 8, 10, 17, 64, 64
_L = len(_LINKAGE)
_C = 2 * _L
_TS = (_H * _W) // 128
_NL = 16
_LH = 10
_ROW_BBOX = _LH * 7
_ROW_H1 = 80
_TR = _ROW_H1 + 64
_BIG = 3.0e38
_MAGIC = 12582912.0


def _row(l, q):
    return (l * 7 + q) if l < _LH else (_ROW_H1 + (l - _LH) * 7 + q)


def _round_ne(x):
    return (x + _MAGIC) - _MAGIC


def _rsqrt_newton(n2):
    i = lax.bitcast_convert_type(n2, jnp.int32)
    i = jnp.int32(0x5F3759DF) - lax.shift_right_arithmetic(i, 1)
    y = lax.bitcast_convert_type(i, jnp.float32)
    for _ in range(3):
        y = y * (1.5 - 0.5 * n2 * y * y)
    return y


def _sc_body(jt_hbm, t_hbm, jv, t_v):
    w = lax.axis_index("c") * 16 + lax.axis_index("s")

    @pl.when(w < 2 * _BS)
    def _():
        b = w // 2
        half = w % 2
        pltpu.sync_copy(jt_hbm.at[b], jv)

        lane = lax.broadcasted_iota(jnp.int32, (_NL,), 0)
        one = jnp.ones((_NL,), jnp.float32)

        tlx = one * _BIG
        tly = one * _BIG
        brx = -one * _BIG
        bry = -one * _BIG
        visp = lane < 0
        for j in range(_J):
            xj = jv[j, 0]
            yj = jv[j, 1]
            vj = jv[j, 2]
            inv = vj <= 0.0
            tlx = jnp.minimum(tlx, jnp.where(inv, _BIG, xj))
            tly = jnp.minimum(tly, jnp.where(inv, _BIG, yj))
            brx = jnp.maximum(brx, jnp.where(inv, -_BIG, xj))
            bry = jnp.maximum(bry, jnp.where(inv, -_BIG, yj))
            visp = visp | (vj > 0.0)

        @pl.when(half == 0)
        def _():
            whx = brx - tlx
            why = bry - tly
            whx = jnp.where(whx < 1.0, 1.0, whx)
            why = jnp.where(why < 1.0, 1.0, why)
            ctx = 0.5 * (brx + tlx)
            cty = 0.5 * (bry + tly)
            whx2 = jnp.maximum(whx, why / _HW_RATIO)
            why2 = jnp.maximum(why, whx / _HW_RATIO)
            exp = jnp.float32(0.5 + _EXPANSION)
            rb = _ROW_BBOX * _NL
            t_v[pl.ds(rb + 0 * _NL, _NL)] = jnp.where(
                visp, _round_ne(ctx - exp * whx2), _BIG)
            t_v[pl.ds(rb + 1 * _NL, _NL)] = jnp.where(
                visp, _round_ne(ctx + exp * whx2), -_BIG)
            t_v[pl.ds(rb + 2 * _NL, _NL)] = jnp.where(
                visp, _round_ne(cty - exp * why2), _BIG)
            t_v[pl.ds(rb + 3 * _NL, _NL)] = jnp.where(
                visp, _round_ne(cty + exp * why2), -_BIG)

        for l, (a, bb) in enumerate(_LINKAGE):
            @pl.when(half == (0 if l < _LH else 1))
            def _(a=a, bb=bb, l=l):
                sx = jv[a, 0]
                sy = jv[a, 1]
                sv = jv[a, 2]
                ex = jv[bb, 0]
                ey = jv[bb, 1]
                ev = jv[bb, 2]
                valid = ((sv > 0.0) & (ev > 0.0) &
                         ((sx != ex) | (sy != ey)))
                vecx = jnp.where(valid, ex - sx, 1.0)
                vecy = jnp.where(valid, ey - sy, 1.0)
                n2 = vecx * vecx + vecy * vecy
                rs = _rsqrt_newton(n2)
                ux = vecx * rs
                uy = vecy * rs
                norm = n2 * rs
                sdot = sx * ux + sy * uy
                scross = sx * uy - sy * ux
                r = _row(l, 0) * _NL

                def st(q, vec):
                    t_v[pl.ds(r + q * _NL, _NL)] = vec

                st(0, ux)
                st(1, uy)
                st(2, sdot - _PDT)
                st(3, jnp.where(valid, sdot + norm + _PDT, -_BIG))
                st(4, scross - _PDT)
                st(5, scross + _PDT)
                invis_l = (sv <= 0.0) | (ev <= 0.0)
                st(6, jnp.where(visp & invis_l, 1.0, 0.0))

        @pl.when(half == 0)
        def _():
            pltpu.sync_copy(t_v.at[pl.ds(0, _ROW_H1 * _NL)],
                            t_hbm.at[b, 0, pl.ds(0, _ROW_H1 * _NL)])

        @pl.when(half == 1)
        def _():
            nw = (_TR - _ROW_H1) * _NL
            pltpu.sync_copy(t_v.at[pl.ds(_ROW_H1 * _NL, nw)],
                            t_hbm.at[b, 0, pl.ds(_ROW_H1 * _NL, nw)])


@functools.lru_cache(maxsize=1)
def _sc_prep():
    return pl.kernel(
        _sc_body,
        out_type=jax.ShapeDtypeStruct((_BS, 1, _TR * _NL), jnp.float32),
        mesh=plsc.VectorSubcoreMesh(core_axis_name="c",
                                    subcore_axis_name="s",
                                    num_cores=1),
        scratch_types=[
            pltpu.VMEM((_J, 3, _NL), jnp.float32),
            pltpu.VMEM((_TR * _NL,), jnp.float32),
        ],
    )


def _tc_body(paf_ref, m_ref, t_ref, out_ref):
    f32 = jnp.float32

    def ts(row, p):
        return t_ref[0, 0, row * _NL + p]

    pix = lax.broadcasted_iota(jnp.int32, (_TS, 128), 0) * 128 + \
        lax.broadcasted_iota(jnp.int32, (_TS, 128), 1)
    yf = (pix // _W).astype(f32)
    xf = (pix % _W).astype(f32)

    bits = jnp.zeros((_TS, 128), jnp.int32)
    for p in range(_P):
        inb = ((xf >= ts(_ROW_BBOX + 0, p)) &
               (xf <= ts(_ROW_BBOX + 1, p)) &
               (yf >= ts(_ROW_BBOX + 2, p)) &
               (yf <= ts(_ROW_BBOX + 3, p)))
        bits = bits + jnp.where(inb, jnp.int32(1 << p), jnp.int32(0))

    mask_t = m_ref[0]
    lacc = jnp.zeros((_TS, 128), f32)
    for l in range(_L):
        count = jnp.zeros((_TS, 128), f32)
        txn = jnp.zeros((_TS, 128), f32)
        tyn = jnp.zeros((_TS, 128), f32)
        for p in range(_P):
            ux_s = ts(_row(l, 0), p)
            uy_s = ts(_row(l, 1), p)
            tt = xf * ux_s + yf * uy_s
            cc = xf * uy_s - yf * ux_s
            m = ((tt >= ts(_row(l, 2), p)) & (tt <= ts(_row(l, 3), p)) &
                 (cc >= ts(_row(l, 4), p)) & (cc <= ts(_row(l, 5), p)))
            count = count + jnp.where(m, 1.0, 0.0)
            txn = txn + jnp.where(m, ux_s, 0.0)
            tyn = tyn + jnp.where(m, uy_s, 0.0)

        act = jnp.int32(0)
        for p in range(_P):
            act = act + jnp.where(ts(_row(l, 6), p) > 0.0,
                                  jnp.int32(1 << p), jnp.int32(0))
        bad = (bits & act) != 0
        paf_lw = jnp.minimum(mask_t, jnp.where(bad, 0.0, 1.0))
        div = jnp.maximum(count, 1.0)
        tx = txn / div
        ty = tyn / div
        lw = jnp.where(count > 0.0, 1.0, paf_lw)
        dx = paf_ref[0, 2 * l] - tx
        dy = paf_ref[0, 2 * l + 1] - ty
        lacc = lacc + (dx * dx + dy * dy) * lw

    out_ref[0, 0] = jnp.full((128,), jnp.sum(lacc) / (_C * _H * _W),
                             jnp.float32)


def kernel(paf_pred, jointsXYV, mask):
    jt = jnp.transpose(jointsXYV, (0, 2, 3, 1))
    jt = jnp.pad(jt, ((0, 0), (0, 0), (0, 0), (0, _NL - _P)),
                 constant_values=-1.0)
    t = _sc_prep()(jt)

    paf2 = paf_pred.reshape(_BS, _C, _TS, 128)
    mask2 = mask.reshape(_BS, _TS, 128)
    out = pl.pallas_call(
        _tc_body,
        grid=(_BS,),
        in_specs=[
            pl.BlockSpec((1, _C, _TS, 128), lambda b: (b, 0, 0, 0)),
            pl.BlockSpec((1, _TS, 128), lambda b: (b, 0, 0)),
            pl.BlockSpec((1, 1, _TR * _NL), lambda b: (b, 0, 0),
                         memory_space=pltpu.SMEM),
        ],
        out_specs=pl.BlockSpec((1, 1, 128), lambda b: (b, 0, 0)),
        out_shape=jax.ShapeDtypeStruct((_BS, 1, 128), jnp.float32),
    )(paf2, mask2, t)
    return out[:, 0, 0]
